# baseline (speedup 1.0000x reference)
import functools

import jax
import jax.numpy as jnp
from jax import lax
from jax.experimental import pallas as pl
from jax.experimental.pallas import tpu as pltpu
from jax.experimental.pallas import tpu_sc as plsc

NUM_FIELDS = 26
EMBED_DIM = 32
BATCH = 16384
VOCAB = 100000

_info = plsc.get_sparse_core_info()
_NC, _NS = _info.num_cores, _info.num_subcores
_NW = _NC * _NS

_CH = 4096
_NCH = BATCH // _CH
_PIECES = ((0, 25088, 25088), (25088, 25088, 25088),
           (50176, 25088, 25088), (75264, 24704, 24736))
_NP = len(_PIECES)
_TAIL0 = 99968
_TOFF = _PIECES[3][1]


def _sc_body(*refs):
    fsS = refs[0]
    Ws = refs[1:1 + NUM_FIELDS]
    tails = refs[1 + NUM_FIELDS]
    out = refs[2 + NUM_FIELDS]
    r = 3 + NUM_FIELDS
    sh_idx = refs[r]
    row0, row1, out_v, idx0, idx1 = refs[r + 1:r + 6]
    rsem0, rsem1, isem0, isem1, osem, ssem = refs[r + 6:r + 12]
    rows = (row0, row1)
    rsems = (rsem0, rsem1)
    idxb = (idx0, idx1)
    isems = (isem0, isem1)

    cid = lax.axis_index("c")
    sid = lax.axis_index("s")
    wid = sid * _NC + cid

    def piece_issue(p, tbl):
        base, words, _ = _PIECES[p]
        pltpu.async_copy(tbl.at[wid, pl.ds(base, words)],
                         rows[p % 2].at[pl.ds(0, words)], rsems[p % 2])

    def tail_issue(fa):
        pltpu.async_copy(
            tails.at[jnp.minimum(fa, NUM_FIELDS - 1), wid],
            rows[1].at[pl.ds(_TOFF, 128)], rsems[1])

    def row_issue_dyn(p, fa, lo):
        for i in range(lo, NUM_FIELDS):
            @pl.when(fa == i)
            def _(i=i):
                piece_issue(p, Ws[i])
        if p == _NP - 1:
            tail_issue(fa)

    def row_wait(p):
        base, words, _ = _PIECES[p]
        pltpu.make_async_copy(Ws[0].at[wid, pl.ds(base, words)],
                              rows[p % 2].at[pl.ds(0, words)],
                              rsems[p % 2]).wait()
        if p == _NP - 1:
            pltpu.make_async_copy(tails.at[0, 0],
                                  rows[1].at[pl.ds(_TOFF, 128)],
                                  rsems[1]).wait()

    def idx_issue(fa, c, slot):
        pltpu.async_copy(fsS.at[fa, pl.ds(c * _CH, _CH)], idxb[slot],
                         isems[slot])

    def idx_wait(slot):
        pltpu.make_async_copy(fsS.at[0, pl.ds(0, _CH)], idxb[slot],
                              isems[slot]).wait()

    def phase(p, fa):
        base, _, span = _PIECES[p]
        buf = rows[p % 2]
        for c in range(_NCH):
            if c == 0:
                idx_issue(fa, 0, 0)
            idx_wait(c % 2)
            if c + 1 < _NCH:
                idx_issue(fa, c + 1, (c + 1) % 2)

            def step(i, _, c=c):
                b = i * 128
                for u in range(8):
                    sl = pl.ds(b + u * 16, 16)
                    osl = pl.ds(c * _CH + b + u * 16, 16)
                    lu = plsc.bitcast(idxb[c % 2][sl] - base, jnp.uint32)
                    cl = plsc.bitcast(
                        jnp.minimum(lu, jnp.uint32(span - 1)), jnp.int32)
                    g = plsc.load_gather(buf, [cl])
                    val = jnp.where(lu < span, g, 0.0)
                    if p == 0:
                        out_v[osl] = val
                    else:
                        plsc.addupdate(out_v.at[osl], val)
                return 0

            lax.fori_loop(0, _CH // 128, step, 0)
            if p == _NP - 1:
                pltpu.async_copy(
                    out_v.at[pl.ds(c * _CH, _CH)],
                    out.at[fa * EMBED_DIM + wid, pl.ds(c * _CH, _CH)], osem)

    piece_issue(0, Ws[0])
    piece_issue(1, Ws[0])

    def field_step(f, _):
        for p in range(_NP):
            row_wait(p)
            phase(p, f)
            if p + 2 < _NP:
                row_issue_dyn(p + 2, f, 0)
            else:
                row_issue_dyn(p + 2 - _NP, f + 1, 1)
        for c in range(_NCH):
            pltpu.make_async_copy(
                out_v.at[pl.ds(c * _CH, _CH)],
                out.at[0, pl.ds(c * _CH, _CH)], osem).wait()
        return 0

    lax.fori_loop(0, NUM_FIELDS, field_step, 0)


@jax.jit
def _cat_embed(fsS, tails, *tables_t):
    mesh = plsc.VectorSubcoreMesh(core_axis_name="c", subcore_axis_name="s")
    k = functools.partial(
        pl.kernel,
        mesh=mesh,
        out_type=jax.ShapeDtypeStruct((NUM_FIELDS * EMBED_DIM, BATCH),
                                      jnp.float32),
        scratch_types=[
            pltpu.VMEM_SHARED((NUM_FIELDS, BATCH), jnp.int32),
            pltpu.VMEM((_PIECES[0][1],), jnp.float32),
            pltpu.VMEM((_PIECES[1][1],), jnp.float32),
            pltpu.VMEM((BATCH,), jnp.float32),
            pltpu.VMEM((_CH,), jnp.int32),
            pltpu.VMEM((_CH,), jnp.int32),
            pltpu.SemaphoreType.DMA,
            pltpu.SemaphoreType.DMA,
            pltpu.SemaphoreType.DMA,
            pltpu.SemaphoreType.DMA,
            pltpu.SemaphoreType.DMA,
            pltpu.SemaphoreType.DMA,
        ],
        compiler_params=pltpu.CompilerParams(use_tc_tiling_on_sc=True,
                                             needs_layout_passes=False),
    )(_sc_body)
    out_t = k(fsS, *tables_t, tails)
    return out_t.T


def kernel(f0, f1, f2, f3, f4, f5, f6, f7, f8, f9, f10, f11, f12, f13,
           f14, f15, f16, f17, f18, f19, f20, f21, f22, f23, f24, f25,
           W0, W1, W2, W3, W4, W5, W6, W7, W8, W9, W10, W11, W12, W13,
           W14, W15, W16, W17, W18, W19, W20, W21, W22, W23, W24, W25):
    fields = [f0, f1, f2, f3, f4, f5, f6, f7, f8, f9, f10, f11, f12, f13,
              f14, f15, f16, f17, f18, f19, f20, f21, f22, f23, f24, f25]
    tables = [W0, W1, W2, W3, W4, W5, W6, W7, W8, W9, W10, W11, W12, W13,
              W14, W15, W16, W17, W18, W19, W20, W21, W22, W23, W24, W25]
    fsS = jnp.stack([jnp.asarray(f, jnp.int32) for f in fields])
    tails = jnp.pad(
        jnp.stack([jnp.transpose(W[_TAIL0:, :]) for W in tables]),
        ((0, 0), (0, 0), (0, 128 - (VOCAB - _TAIL0))))
    tables_t = [jnp.transpose(W) for W in tables]
    return _cat_embed(fsS, tails, *tables_t)


# speedup vs baseline: 2.4132x; 2.4132x over previous
import functools

import jax
import jax.numpy as jnp
from jax import lax
from jax.experimental import pallas as pl
from jax.experimental.pallas import tpu as pltpu
from jax.experimental.pallas import tpu_sc as plsc

NUM_FIELDS = 26
EMBED_DIM = 32
BATCH = 16384
VOCAB = 100000

_info = plsc.get_sparse_core_info()
_NC, _NS = _info.num_cores, _info.num_subcores
_NW = _NC * _NS

_CH = 8192


def _sc_body(*refs):
    fs = refs[:NUM_FIELDS]
    Ws = refs[NUM_FIELDS:2 * NUM_FIELDS]
    out = refs[2 * NUM_FIELDS]
    r = 2 * NUM_FIELDS + 1
    row_v, idx_v, out_v, rsem, isem, osem = refs[r:r + 6]

    wid = lax.axis_index("s") * _NC + lax.axis_index("c")

    prev_store = None
    for f in range(NUM_FIELDS):
        rd = pltpu.async_copy(Ws[f].at[wid], row_v, rsem)
        for c in range(BATCH // _CH):
            idm = pltpu.async_copy(fs[f].at[pl.ds(c * _CH, _CH)], idx_v, isem)
            idm.wait()
            if c == 0:
                rd.wait()
            if prev_store is not None:
                prev_store.wait()

            def gather_step(i, _):
                b = i * 128
                for u in range(8):
                    iv = idx_v[pl.ds(b + u * 16, 16)]
                    out_v[pl.ds(b + u * 16, 16)] = plsc.load_gather(
                        row_v, [iv])
                return 0

            lax.fori_loop(0, _CH // 128, gather_step, 0)
            prev_store = pltpu.async_copy(
                out_v, out.at[f * EMBED_DIM + wid, pl.ds(c * _CH, _CH)], osem)
    prev_store.wait()


@jax.jit
def _cat_embed(*args):
    mesh = plsc.VectorSubcoreMesh(core_axis_name="c", subcore_axis_name="s")
    k = functools.partial(
        pl.kernel,
        mesh=mesh,
        out_type=jax.ShapeDtypeStruct((NUM_FIELDS * EMBED_DIM, BATCH),
                                      jnp.float32),
        scratch_types=[
            pltpu.VMEM((VOCAB,), jnp.float32),
            pltpu.VMEM((_CH,), jnp.int32),
            pltpu.VMEM((_CH,), jnp.float32),
            pltpu.SemaphoreType.DMA,
            pltpu.SemaphoreType.DMA,
            pltpu.SemaphoreType.DMA,
        ],
        compiler_params=pltpu.CompilerParams(use_tc_tiling_on_sc=True,
                                             needs_layout_passes=False),
    )(_sc_body)
    out_t = k(*args)
    return out_t.T


def kernel(f0, f1, f2, f3, f4, f5, f6, f7, f8, f9, f10, f11, f12, f13,
           f14, f15, f16, f17, f18, f19, f20, f21, f22, f23, f24, f25,
           W0, W1, W2, W3, W4, W5, W6, W7, W8, W9, W10, W11, W12, W13,
           W14, W15, W16, W17, W18, W19, W20, W21, W22, W23, W24, W25):
    fields = [f0, f1, f2, f3, f4, f5, f6, f7, f8, f9, f10, f11, f12, f13,
              f14, f15, f16, f17, f18, f19, f20, f21, f22, f23, f24, f25]
    tables = [W0, W1, W2, W3, W4, W5, W6, W7, W8, W9, W10, W11, W12, W13,
              W14, W15, W16, W17, W18, W19, W20, W21, W22, W23, W24, W25]
    fields = [jnp.asarray(f, jnp.int32) for f in fields]
    tables_t = [jnp.transpose(W) for W in tables]
    return _cat_embed(*fields, *tables_t)
